# baseline (speedup 1.0000x reference)
import functools

import jax
import jax.numpy as jnp
from jax import lax
from jax.experimental import pallas as pl
from jax.experimental.pallas import tpu as pltpu
from jax.experimental.pallas import tpu_sc as plsc

N = 10000
E = 160000
H = 128
f32 = jnp.float32
bf16 = jnp.bfloat16


BN = 1000
BE = 640
NCHUNK = E // 128


BP = 2000


def _proj_body(x_ref, w_ref, o_ref):
    o_ref[0] = jnp.dot(x_ref[...], w_ref[0],
                       preferred_element_type=f32).astype(bf16)


def _proj(x, w_stack):
    k = w_stack.shape[0]
    return pl.pallas_call(
        _proj_body,
        grid=(N // BP, k),
        in_specs=[
            pl.BlockSpec((BP, H), lambda i, j: (i, 0)),
            pl.BlockSpec((1, H, H), lambda i, j: (j, 0, 0)),
        ],
        out_specs=pl.BlockSpec((1, BP, H), lambda i, j: (j, i, 0)),
        out_shape=jax.ShapeDtypeStruct((k, N, H), bf16),
    )(x, w_stack)


def _msg_body(g_ref, ea_ref, w1_ref, b1_ref, w2_ref, b2_ref, o_ref):
    h = g_ref[0].astype(f32) + jnp.dot(ea_ref[...].astype(bf16), w1_ref[...],
                                       preferred_element_type=f32)
    h = jnp.maximum(h + b1_ref[...], 0.0)
    o_ref[...] = jnp.dot(h.astype(bf16), w2_ref[...],
                         preferred_element_type=f32) + b2_ref[...]


def _msg(g9, kplane, ea, w1, b1, w2, b2):
    return pl.pallas_call(
        _msg_body,
        grid=(E // BE,),
        in_specs=[
            pl.BlockSpec((1, BE, H), lambda i: (kplane, i, 0)),
            pl.BlockSpec((BE, H), lambda i: (i, 0)),
            pl.BlockSpec((H, H), lambda i: (0, 0)),
            pl.BlockSpec((1, H), lambda i: (0, 0)),
            pl.BlockSpec((H, H), lambda i: (0, 0)),
            pl.BlockSpec((1, H), lambda i: (0, 0)),
        ],
        out_specs=pl.BlockSpec((BE, H), lambda i: (i, 0)),
        out_shape=jax.ShapeDtypeStruct((E, H), f32),
    )(g9, ea, w1, b1, w2, b2)


def _edge_body(gs_ref, gd_ref, ea_ref, w1_ref, b1_ref, w2_ref, b2_ref, o_ref):
    ea = ea_ref[...]
    h = (gs_ref[0] + gd_ref[0]).astype(f32) + jnp.dot(
        ea.astype(bf16), w1_ref[...], preferred_element_type=f32)
    h = jnp.maximum(h + b1_ref[...], 0.0)
    o_ref[...] = ea + jnp.dot(h.astype(bf16), w2_ref[...],
                              preferred_element_type=f32) + b2_ref[...]


def _edge(g9, ks, kd, ea, w1, b1, w2, b2):
    return pl.pallas_call(
        _edge_body,
        grid=(E // BE,),
        in_specs=[
            pl.BlockSpec((1, BE, H), lambda i: (ks, i, 0)),
            pl.BlockSpec((1, BE, H), lambda i: (kd, i, 0)),
            pl.BlockSpec((BE, H), lambda i: (i, 0)),
            pl.BlockSpec((H, H), lambda i: (0, 0)),
            pl.BlockSpec((1, H), lambda i: (0, 0)),
            pl.BlockSpec((H, H), lambda i: (0, 0)),
            pl.BlockSpec((1, H), lambda i: (0, 0)),
        ],
        out_specs=pl.BlockSpec((BE, H), lambda i: (i, 0)),
        out_shape=jax.ShapeDtypeStruct((E, H), f32),
    )(g9, g9, ea, w1, b1, w2, b2)


def _node_body(x_ref, at_ref, as_ref, av_ref, w_ref, b1_ref, w2_ref, b2_ref, o_ref):
    x = x_ref[...]
    u = jnp.dot(x, w_ref[0], preferred_element_type=f32)
    u += jnp.dot(at_ref[0] + at_ref[1], w_ref[1], preferred_element_type=f32)
    u += jnp.dot(as_ref[0] + as_ref[1], w_ref[2], preferred_element_type=f32)
    u += jnp.dot(av_ref[0] + av_ref[1], w_ref[3], preferred_element_type=f32)
    u = jnp.maximum(u + b1_ref[...], 0.0)
    o_ref[...] = x + jnp.dot(u, w2_ref[...], preferred_element_type=f32) + b2_ref[...]


def _node(x, agg_t, agg_s, agg_v, w1s, b1, w2, b2):
    return pl.pallas_call(
        _node_body,
        grid=(N // BN,),
        in_specs=[
            pl.BlockSpec((BN, H), lambda i: (i, 0)),
            pl.BlockSpec((2, BN, H), lambda i: (0, i, 0)),
            pl.BlockSpec((2, BN, H), lambda i: (0, i, 0)),
            pl.BlockSpec((2, BN, H), lambda i: (0, i, 0)),
            pl.BlockSpec((4, H, H), lambda i: (0, 0, 0)),
            pl.BlockSpec((1, H), lambda i: (0, 0)),
            pl.BlockSpec((H, H), lambda i: (0, 0)),
            pl.BlockSpec((1, H), lambda i: (0, 0)),
        ],
        out_specs=pl.BlockSpec((BN, H), lambda i: (i, 0)),
        out_shape=jax.ShapeDtypeStruct((N, H), f32),
    )(x, agg_t, agg_s, agg_v, w1s, b1, w2, b2)


def _headproj_body(x_ref, w_ref, b_ref, o_ref):
    o_ref[...] = jnp.dot(x_ref[...], w_ref[...], preferred_element_type=f32) + b_ref[...]


def _headproj(x, w, b):
    return pl.pallas_call(
        _headproj_body,
        grid=(N // BN,),
        in_specs=[
            pl.BlockSpec((BN, H), lambda i: (i, 0)),
            pl.BlockSpec((H, 8), lambda i: (0, 0)),
            pl.BlockSpec((1, 8), lambda i: (0, 0)),
        ],
        out_specs=pl.BlockSpec((BN, 8), lambda i: (i, 0)),
        out_shape=jax.ShapeDtypeStruct((N, 8), f32),
    )(x, w, b)


def _eterm(ea, w, b):
    return pl.pallas_call(
        _headproj_body,
        grid=(E // BE,),
        in_specs=[
            pl.BlockSpec((BE, H), lambda i: (i, 0)),
            pl.BlockSpec((H, 8), lambda i: (0, 0)),
            pl.BlockSpec((1, 8), lambda i: (0, 0)),
        ],
        out_specs=pl.BlockSpec((BE, 8), lambda i: (i, 0)),
        out_shape=jax.ShapeDtypeStruct((E, 8), f32),
    )(ea, w, b)


@functools.lru_cache(maxsize=1)
def _mesh():
    return plsc.VectorSubcoreMesh(core_axis_name="core", subcore_axis_name="subcore")


def _gather_rows(table, idx2d, m):
    @functools.partial(
        pl.kernel,
        out_type=jax.ShapeDtypeStruct((m, H), table.dtype),
        mesh=_mesh(),
        compiler_params=pltpu.CompilerParams(use_tc_tiling_on_sc=False),
    )
    def k(table_hbm, i_hbm, o_hbm):
        def body(i_vmem, o_vmem):
            pltpu.sync_copy(table_hbm.at[i_vmem.at[0]], o_vmem)

        pltpu.emit_pipeline(
            body,
            grid=(m // 128,),
            in_specs=[pl.BlockSpec((1, 128), lambda i: (0, i))],
            out_specs=[pl.BlockSpec((128, H), lambda i: (i, 0))],
            core_axis_name=("core", "subcore"),
            dimension_semantics=(pltpu.PARALLEL,),
        )(i_hbm, o_hbm)

    return k(table, idx2d)


def _segsum(msg, dstm, zrows):
    @functools.partial(
        pl.kernel,
        out_type=jax.ShapeDtypeStruct((2, N, H), f32),
        mesh=_mesh(),
        compiler_params=pltpu.CompilerParams(use_tc_tiling_on_sc=False),
        scratch_types=[
            pltpu.VMEM_SHARED((N, H), f32),
            pltpu.VMEM((1, 128), jnp.int32),
            pltpu.VMEM((128, H), f32),
        ],
    )
    def k(msg_hbm, dstm_hbm, z_hbm, out_hbm, acc, idx_b, rows_b):
        cid = lax.axis_index("core")
        sid = lax.axis_index("subcore")
        stripe = N // 16
        pltpu.sync_copy(z_hbm, acc.at[pl.ds(sid * stripe, stripe)])
        plsc.subcore_barrier()
        half = NCHUNK // 2
        base = cid * half + sid
        nk = jnp.where(sid == 0, (half + 15) // 16, half // 16)

        def step(kk, _):
            c = base + kk * 16
            pltpu.sync_copy(dstm_hbm.at[pl.ds(c, 1)], idx_b)
            pltpu.sync_copy(msg_hbm.at[pl.ds(c * 128, 128)], rows_b)
            pltpu.sync_copy(rows_b, acc.at[idx_b.at[0]], add=True)
            return 0

        lax.fori_loop(0, nk, step, 0)
        plsc.subcore_barrier()
        pltpu.sync_copy(
            acc.at[pl.ds(sid * stripe, stripe)],
            out_hbm.at[cid, pl.ds(sid * stripe, stripe)],
        )

    return k(msg, dstm, zrows)


def _edge_heads(srcm_t, dstm_t, et_t, tab_st, tab_dt,
                srcm_v, dstm_v, et_v, tab_sv, tab_dv):
    @functools.partial(
        pl.kernel,
        out_type=jax.ShapeDtypeStruct((2, NCHUNK, 128), f32),
        mesh=_mesh(),
        compiler_params=pltpu.CompilerParams(use_tc_tiling_on_sc=False,
                                             needs_layout_passes=False),
        scratch_types=[
            pltpu.VMEM((N,), f32),
            pltpu.VMEM((N,), f32),
            pltpu.VMEM((N,), f32),
            pltpu.VMEM((N,), f32),
            pltpu.VMEM((1, 128), jnp.int32),
            pltpu.VMEM((1, 128), jnp.int32),
            pltpu.VMEM((1, 128), f32),
            pltpu.VMEM((1, 128), f32),
        ],
    )
    def k(srcm_t_hbm, dstm_t_hbm, et_t_hbm, tab_st_hbm, tab_dt_hbm,
          srcm_v_hbm, dstm_v_hbm, et_v_hbm, tab_sv_hbm, tab_dv_hbm,
          out_hbm, st_v, dt_v, sv_v, dv_v, is_b, id_b, e_b, o_b):
        cid = lax.axis_index("core")
        sid = lax.axis_index("subcore")
        wid = sid * 2 + cid
        pltpu.sync_copy(tab_st_hbm, st_v)
        pltpu.sync_copy(tab_dt_hbm, dt_v)
        pltpu.sync_copy(tab_sv_hbm, sv_v)
        pltpu.sync_copy(tab_dv_hbm, dv_v)
        nk = jnp.where(wid < NCHUNK % 32, NCHUNK // 32 + 1, NCHUNK // 32)

        for ty, (s_hbm, d_hbm, e_hbm, stab, dtab) in enumerate(
            [(srcm_t_hbm, dstm_t_hbm, et_t_hbm, st_v, dt_v),
             (srcm_v_hbm, dstm_v_hbm, et_v_hbm, sv_v, dv_v)]):

            def step(kk, _):
                c = wid + kk * 32
                pltpu.sync_copy(s_hbm.at[pl.ds(c, 1)], is_b)
                pltpu.sync_copy(d_hbm.at[pl.ds(c, 1)], id_b)
                pltpu.sync_copy(e_hbm.at[pl.ds(c, 1)], e_b)
                for j in range(8):
                    sl = pl.ds(j * 16, 16)
                    vs = plsc.load_gather(stab, [is_b[0, sl]])
                    vd = plsc.load_gather(dtab, [id_b[0, sl]])
                    o_b[0, sl] = vs + vd + e_b[0, sl]
                pltpu.sync_copy(o_b, out_hbm.at[ty, pl.ds(c, 1)])
                return 0

            lax.fori_loop(0, nk, step, 0)

    return k(srcm_t, dstm_t, et_t, tab_st, tab_dt,
             srcm_v, dstm_v, et_v, tab_sv, tab_dv)


def kernel(x, edge_attr_t, edge_attr_s, edge_attr_v, params,
           edge_index_t, edge_index_s, edge_index_v):
    p = params
    ea = {'t': edge_attr_t, 's': edge_attr_s, 'v': edge_attr_v}
    ei = {'t': edge_index_t, 's': edge_index_s, 'v': edge_index_v}
    ets = ['t', 's', 'v']

    w9 = jnp.stack(
        [p['Wm_' + et + '1'][:H] for et in ets]
        + sum([[p['We_' + et + '1'][:H], p['We_' + et + '1'][H:2 * H]] for et in ets], []),
        axis=0)
    wn1s = jnp.stack([p['Wn1'][i * H:(i + 1) * H] for i in range(4)], axis=0)

    src = {et: ei[et][0] for et in ets}
    dst = {et: ei[et][1] for et in ets}
    gidx = jnp.concatenate([
        src['t'], src['s'] + N, src['v'] + 2 * N,
        src['t'] + 3 * N, dst['t'] + 4 * N,
        src['s'] + 5 * N, dst['s'] + 6 * N,
        src['v'] + 7 * N, dst['v'] + 8 * N,
    ]).reshape(1, 9 * E)
    dstm = {et: dst[et].reshape(NCHUNK, 128) for et in ets}
    zrows = jnp.zeros((N // 16, H), f32)

    b1 = {et: p['bm_' + et + '1'].reshape(1, H) for et in ets}
    b2 = {et: p['bm_' + et + '2'].reshape(1, H) for et in ets}
    be1 = {et: p['be_' + et + '1'].reshape(1, H) for et in ets}
    be2 = {et: p['be_' + et + '2'].reshape(1, H) for et in ets}
    bn1 = p['bn1'].reshape(1, H)
    bn2 = p['bn2'].reshape(1, H)

    for _ in range(2):
        proj = _proj(x, w9)
        g9 = _gather_rows(proj.reshape(9 * N, H), gidx, 9 * E).reshape(9, E, H)
        aggs = []
        for i, et in enumerate(ets):
            msg = _msg(g9, i, ea[et], p['Wm_' + et + '1'][H:].astype(bf16),
                       b1[et], p['Wm_' + et + '2'].astype(bf16), b2[et])
            aggs.append(_segsum(msg, dstm[et], zrows))
        x_new = _node(x, aggs[0], aggs[1], aggs[2], wn1s, bn1, p['Wn2'], bn2)
        for i, et in enumerate(ets):
            ea[et] = _edge(g9, 3 + 2 * i, 4 + 2 * i, ea[et],
                           p['We_' + et + '1'][2 * H:].astype(bf16), be1[et],
                           p['We_' + et + '2'].astype(bf16), be2[et])
        x = x_new

    wh = jnp.concatenate([
        p['Wout_n'], p['Wout_et'][:H], p['Wout_et'][H:2 * H],
        p['Wout_ev'][:H], p['Wout_ev'][H:2 * H],
        jnp.zeros((H, 3), f32),
    ], axis=1)
    bh = jnp.concatenate([p['bout_n'], jnp.zeros((7,), f32)]).reshape(1, 8)
    hp = _headproj(x, wh, bh)
    node_pred = hp[:, 0:1]

    et_t = _eterm(ea['t'], jnp.pad(p['Wout_et'][2 * H:], ((0, 0), (0, 7))),
                  jnp.pad(p['bout_et'], (0, 7)).reshape(1, 8))[:, 0]
    et_v = _eterm(ea['v'], jnp.pad(p['Wout_ev'][2 * H:], ((0, 0), (0, 7))),
                  jnp.pad(p['bout_ev'], (0, 7)).reshape(1, 8))[:, 0]

    ep = _edge_heads(
        src['t'].reshape(NCHUNK, 128), dst['t'].reshape(NCHUNK, 128),
        et_t.reshape(NCHUNK, 128), hp[:, 1], hp[:, 2],
        src['v'].reshape(NCHUNK, 128), dst['v'].reshape(NCHUNK, 128),
        et_v.reshape(NCHUNK, 128), hp[:, 3], hp[:, 4])
    ep_t = ep[0].reshape(E, 1)
    ep_v = ep[1].reshape(E, 1)
    return (node_pred, ep_t, ep_v)


# speedup vs baseline: 1.5839x; 1.5839x over previous
import functools

import jax
import jax.numpy as jnp
from jax import lax
from jax.experimental import pallas as pl
from jax.experimental.pallas import tpu as pltpu
from jax.experimental.pallas import tpu_sc as plsc

N = 10000
E = 160000
H = 128
f32 = jnp.float32
bf16 = jnp.bfloat16


BN = 1000
BE = 640
NCHUNK = E // 128


BP = 2000


def _proj_body(x_ref, w_ref, o_ref):
    o_ref[0] = jnp.dot(x_ref[...], w_ref[0], preferred_element_type=f32)


def _proj(x, w_stack):
    k = w_stack.shape[0]
    return pl.pallas_call(
        _proj_body,
        grid=(N // BP, k),
        in_specs=[
            pl.BlockSpec((BP, H), lambda i, j: (i, 0)),
            pl.BlockSpec((1, H, H), lambda i, j: (j, 0, 0)),
        ],
        out_specs=pl.BlockSpec((1, BP, H), lambda i, j: (j, i, 0)),
        out_shape=jax.ShapeDtypeStruct((k, N, H), f32),
    )(x, w_stack)


def _msg_body(g_ref, ea_ref, w1_ref, b1_ref, w2_ref, b2_ref, o_ref):
    h = g_ref[0].astype(f32) + jnp.dot(ea_ref[...].astype(bf16), w1_ref[...],
                                       preferred_element_type=f32)
    h = jnp.maximum(h + b1_ref[...], 0.0)
    o_ref[...] = jnp.dot(h.astype(bf16), w2_ref[...],
                         preferred_element_type=f32) + b2_ref[...]


def _msg(g9, kplane, ea, w1, b1, w2, b2):
    return pl.pallas_call(
        _msg_body,
        grid=(E // BE,),
        in_specs=[
            pl.BlockSpec((1, BE, H), lambda i: (kplane, i, 0)),
            pl.BlockSpec((BE, H), lambda i: (i, 0)),
            pl.BlockSpec((H, H), lambda i: (0, 0)),
            pl.BlockSpec((1, H), lambda i: (0, 0)),
            pl.BlockSpec((H, H), lambda i: (0, 0)),
            pl.BlockSpec((1, H), lambda i: (0, 0)),
        ],
        out_specs=pl.BlockSpec((BE, H), lambda i: (i, 0)),
        out_shape=jax.ShapeDtypeStruct((EPAD, H), f32),
    )(g9, ea, w1, b1, w2, b2)


def _edge_body(gs_ref, gd_ref, ea_ref, w1_ref, b1_ref, w2_ref, b2_ref, o_ref):
    ea = ea_ref[...]
    h = (gs_ref[0] + gd_ref[0]).astype(f32) + jnp.dot(
        ea.astype(bf16), w1_ref[...], preferred_element_type=f32)
    h = jnp.maximum(h + b1_ref[...], 0.0)
    o_ref[...] = ea + jnp.dot(h.astype(bf16), w2_ref[...],
                              preferred_element_type=f32) + b2_ref[...]


def _edge(g9, ks, kd, ea, w1, b1, w2, b2):
    return pl.pallas_call(
        _edge_body,
        grid=(E // BE,),
        in_specs=[
            pl.BlockSpec((1, BE, H), lambda i: (ks, i, 0)),
            pl.BlockSpec((1, BE, H), lambda i: (kd, i, 0)),
            pl.BlockSpec((BE, H), lambda i: (i, 0)),
            pl.BlockSpec((H, H), lambda i: (0, 0)),
            pl.BlockSpec((1, H), lambda i: (0, 0)),
            pl.BlockSpec((H, H), lambda i: (0, 0)),
            pl.BlockSpec((1, H), lambda i: (0, 0)),
        ],
        out_specs=pl.BlockSpec((BE, H), lambda i: (i, 0)),
        out_shape=jax.ShapeDtypeStruct((E, H), f32),
    )(g9, g9, ea, w1, b1, w2, b2)


def _node_body(x_ref, at_ref, as_ref, av_ref, w_ref, b1_ref, w2_ref, b2_ref, o_ref):
    x = x_ref[...]
    u = jnp.dot(x, w_ref[0], preferred_element_type=f32)
    u += jnp.dot(at_ref[0] + at_ref[1], w_ref[1], preferred_element_type=f32)
    u += jnp.dot(as_ref[0] + as_ref[1], w_ref[2], preferred_element_type=f32)
    u += jnp.dot(av_ref[0] + av_ref[1], w_ref[3], preferred_element_type=f32)
    u = jnp.maximum(u + b1_ref[...], 0.0)
    o_ref[...] = x + jnp.dot(u, w2_ref[...], preferred_element_type=f32) + b2_ref[...]


def _node(x, agg_t, agg_s, agg_v, w1s, b1, w2, b2):
    return pl.pallas_call(
        _node_body,
        grid=(N // BN,),
        in_specs=[
            pl.BlockSpec((BN, H), lambda i: (i, 0)),
            pl.BlockSpec((2, BN, H), lambda i: (0, i, 0)),
            pl.BlockSpec((2, BN, H), lambda i: (0, i, 0)),
            pl.BlockSpec((2, BN, H), lambda i: (0, i, 0)),
            pl.BlockSpec((4, H, H), lambda i: (0, 0, 0)),
            pl.BlockSpec((1, H), lambda i: (0, 0)),
            pl.BlockSpec((H, H), lambda i: (0, 0)),
            pl.BlockSpec((1, H), lambda i: (0, 0)),
        ],
        out_specs=pl.BlockSpec((BN, H), lambda i: (i, 0)),
        out_shape=jax.ShapeDtypeStruct((N, H), f32),
    )(x, agg_t, agg_s, agg_v, w1s, b1, w2, b2)


def _headproj_body(x_ref, w_ref, b_ref, o_ref):
    o_ref[...] = jnp.dot(x_ref[...], w_ref[...], preferred_element_type=f32) + b_ref[...]


def _headproj(x, w, b):
    return pl.pallas_call(
        _headproj_body,
        grid=(N // BN,),
        in_specs=[
            pl.BlockSpec((BN, H), lambda i: (i, 0)),
            pl.BlockSpec((H, 8), lambda i: (0, 0)),
            pl.BlockSpec((1, 8), lambda i: (0, 0)),
        ],
        out_specs=pl.BlockSpec((BN, 8), lambda i: (i, 0)),
        out_shape=jax.ShapeDtypeStruct((N, 8), f32),
    )(x, w, b)


def _eterm(ea, w, b):
    return pl.pallas_call(
        _headproj_body,
        grid=(E // BE,),
        in_specs=[
            pl.BlockSpec((BE, H), lambda i: (i, 0)),
            pl.BlockSpec((H, 8), lambda i: (0, 0)),
            pl.BlockSpec((1, 8), lambda i: (0, 0)),
        ],
        out_specs=pl.BlockSpec((BE, 8), lambda i: (i, 0)),
        out_shape=jax.ShapeDtypeStruct((E, 8), f32),
    )(ea, w, b)


@functools.lru_cache(maxsize=1)
def _mesh():
    return plsc.VectorSubcoreMesh(core_axis_name="core", subcore_axis_name="subcore")


def _gather_rows(table, idx2d, m):
    @functools.partial(
        pl.kernel,
        out_type=jax.ShapeDtypeStruct((m, H), table.dtype),
        mesh=_mesh(),
        compiler_params=pltpu.CompilerParams(use_tc_tiling_on_sc=False),
    )
    def k(table_hbm, i_hbm, o_hbm):
        def body(i_vmem, o_vmem):
            pltpu.sync_copy(table_hbm.at[i_vmem.at[0]], o_vmem)

        pltpu.emit_pipeline(
            body,
            grid=(m // 128,),
            in_specs=[pl.BlockSpec((1, 128), lambda i: (0, i))],
            out_specs=[pl.BlockSpec((128, H), lambda i: (i, 0))],
            core_axis_name=("core", "subcore"),
            dimension_semantics=(pltpu.PARALLEL,),
        )(i_hbm, o_hbm)

    return k(table, idx2d)


NCHUNK_PAD = 1280
EPAD = NCHUNK_PAD * 128
ACC_N = N + 16


def _segsum(msg, dstm, zrows):
    @functools.partial(
        pl.kernel,
        out_type=jax.ShapeDtypeStruct((2, N, H), f32),
        mesh=_mesh(),
        compiler_params=pltpu.CompilerParams(use_tc_tiling_on_sc=False),
        scratch_types=[
            pltpu.VMEM_SHARED((ACC_N, H), f32),
            pltpu.VMEM((2, 1, 128), jnp.int32),
            pltpu.VMEM((2, 128, H), f32),
            pltpu.SemaphoreType.DMA,
            pltpu.SemaphoreType.DMA,
            pltpu.SemaphoreType.DMA,
            pltpu.SemaphoreType.DMA,
        ],
    )
    def k(msg_hbm, dstm_hbm, z_hbm, out_hbm, acc, idx_b, rows_b,
          si0, si1, sr0, sr1):
        cid = lax.axis_index("core")
        sid = lax.axis_index("subcore")
        zstripe = ACC_N // 16
        pltpu.sync_copy(z_hbm, acc.at[pl.ds(sid * zstripe, zstripe)])
        plsc.subcore_barrier()
        base = (cid * 16 + sid) * 40
        sems = (si0, si1)
        semr = (sr0, sr1)

        def issue(c, slot):
            pltpu.async_copy(dstm_hbm.at[pl.ds(c, 1)], idx_b.at[slot],
                             sems[slot])
            pltpu.async_copy(msg_hbm.at[pl.ds(c * 128, 128)], rows_b.at[slot],
                             semr[slot])

        def drain_scatter(c_next, slot, more):
            pltpu.make_async_copy(dstm_hbm.at[pl.ds(0, 1)], idx_b.at[slot],
                                  sems[slot]).wait()
            pltpu.make_async_copy(msg_hbm.at[pl.ds(0, 128)], rows_b.at[slot],
                                  semr[slot]).wait()
            pltpu.sync_copy(rows_b.at[slot], acc.at[idx_b.at[slot, 0]],
                            add=True)

            @pl.when(more)
            def _():
                issue(c_next, slot)

        issue(base, 0)
        issue(base + 1, 1)

        def pair(pp, _):
            drain_scatter(base + 2 * pp + 2, 0, pp < 19)
            drain_scatter(base + 2 * pp + 3, 1, pp < 19)
            return 0

        lax.fori_loop(0, 20, pair, 0)
        plsc.subcore_barrier()
        stripe = N // 16
        pltpu.sync_copy(
            acc.at[pl.ds(sid * stripe, stripe)],
            out_hbm.at[cid, pl.ds(sid * stripe, stripe)],
        )

    return k(msg, dstm, zrows)


def _edge_heads(srcm_t, dstm_t, et_t, tab_st, tab_dt,
                srcm_v, dstm_v, et_v, tab_sv, tab_dv):
    @functools.partial(
        pl.kernel,
        out_type=jax.ShapeDtypeStruct((2, NCHUNK, 128), f32),
        mesh=_mesh(),
        compiler_params=pltpu.CompilerParams(use_tc_tiling_on_sc=False,
                                             needs_layout_passes=False),
        scratch_types=[
            pltpu.VMEM((N,), f32),
            pltpu.VMEM((N,), f32),
            pltpu.VMEM((N,), f32),
            pltpu.VMEM((N,), f32),
            pltpu.VMEM((1, 128), jnp.int32),
            pltpu.VMEM((1, 128), jnp.int32),
            pltpu.VMEM((1, 128), f32),
            pltpu.VMEM((1, 128), f32),
        ],
    )
    def k(srcm_t_hbm, dstm_t_hbm, et_t_hbm, tab_st_hbm, tab_dt_hbm,
          srcm_v_hbm, dstm_v_hbm, et_v_hbm, tab_sv_hbm, tab_dv_hbm,
          out_hbm, st_v, dt_v, sv_v, dv_v, is_b, id_b, e_b, o_b):
        cid = lax.axis_index("core")
        sid = lax.axis_index("subcore")
        wid = sid * 2 + cid
        pltpu.sync_copy(tab_st_hbm, st_v)
        pltpu.sync_copy(tab_dt_hbm, dt_v)
        pltpu.sync_copy(tab_sv_hbm, sv_v)
        pltpu.sync_copy(tab_dv_hbm, dv_v)
        nk = jnp.where(wid < NCHUNK % 32, NCHUNK // 32 + 1, NCHUNK // 32)

        for ty, (s_hbm, d_hbm, e_hbm, stab, dtab) in enumerate(
            [(srcm_t_hbm, dstm_t_hbm, et_t_hbm, st_v, dt_v),
             (srcm_v_hbm, dstm_v_hbm, et_v_hbm, sv_v, dv_v)]):

            def step(kk, _):
                c = wid + kk * 32
                pltpu.sync_copy(s_hbm.at[pl.ds(c, 1)], is_b)
                pltpu.sync_copy(d_hbm.at[pl.ds(c, 1)], id_b)
                pltpu.sync_copy(e_hbm.at[pl.ds(c, 1)], e_b)
                for j in range(8):
                    sl = pl.ds(j * 16, 16)
                    vs = plsc.load_gather(stab, [is_b[0, sl]])
                    vd = plsc.load_gather(dtab, [id_b[0, sl]])
                    o_b[0, sl] = vs + vd + e_b[0, sl]
                pltpu.sync_copy(o_b, out_hbm.at[ty, pl.ds(c, 1)])
                return 0

            lax.fori_loop(0, nk, step, 0)

    return k(srcm_t, dstm_t, et_t, tab_st, tab_dt,
             srcm_v, dstm_v, et_v, tab_sv, tab_dv)


def kernel(x, edge_attr_t, edge_attr_s, edge_attr_v, params,
           edge_index_t, edge_index_s, edge_index_v):
    p = params
    ea = {'t': edge_attr_t, 's': edge_attr_s, 'v': edge_attr_v}
    ei = {'t': edge_index_t, 's': edge_index_s, 'v': edge_index_v}
    ets = ['t', 's', 'v']

    w9 = jnp.stack(
        [p['Wm_' + et + '1'][:H] for et in ets]
        + sum([[p['We_' + et + '1'][:H], p['We_' + et + '1'][H:2 * H]] for et in ets], []),
        axis=0)
    wn1s = jnp.stack([p['Wn1'][i * H:(i + 1) * H] for i in range(4)], axis=0)

    src = {et: ei[et][0] for et in ets}
    dst = {et: ei[et][1] for et in ets}
    gidx = jnp.concatenate([
        src['t'], src['s'] + N, src['v'] + 2 * N,
        src['t'] + 3 * N, dst['t'] + 4 * N,
        src['s'] + 5 * N, dst['s'] + 6 * N,
        src['v'] + 7 * N, dst['v'] + 8 * N,
    ]).reshape(1, 9 * E)
    pad = jnp.full((EPAD - E,), N, jnp.int32)
    dstm = {et: jnp.concatenate([dst[et], pad]).reshape(NCHUNK_PAD, 128)
            for et in ets}
    zrows = jnp.zeros((ACC_N // 16, H), f32)

    b1 = {et: p['bm_' + et + '1'].reshape(1, H) for et in ets}
    b2 = {et: p['bm_' + et + '2'].reshape(1, H) for et in ets}
    be1 = {et: p['be_' + et + '1'].reshape(1, H) for et in ets}
    be2 = {et: p['be_' + et + '2'].reshape(1, H) for et in ets}
    bn1 = p['bn1'].reshape(1, H)
    bn2 = p['bn2'].reshape(1, H)

    for _ in range(2):
        proj = _proj(x, w9)
        g9 = _gather_rows(proj.reshape(9 * N, H), gidx, 9 * E).reshape(9, E, H)
        aggs = []
        for i, et in enumerate(ets):
            msg = _msg(g9, i, ea[et], p['Wm_' + et + '1'][H:].astype(bf16),
                       b1[et], p['Wm_' + et + '2'].astype(bf16), b2[et])
            aggs.append(_segsum(msg, dstm[et], zrows))
        x_new = _node(x, aggs[0], aggs[1], aggs[2], wn1s, bn1, p['Wn2'], bn2)
        for i, et in enumerate(ets):
            ea[et] = _edge(g9, 3 + 2 * i, 4 + 2 * i, ea[et],
                           p['We_' + et + '1'][2 * H:].astype(bf16), be1[et],
                           p['We_' + et + '2'].astype(bf16), be2[et])
        x = x_new

    wh = jnp.concatenate([
        p['Wout_n'], p['Wout_et'][:H], p['Wout_et'][H:2 * H],
        p['Wout_ev'][:H], p['Wout_ev'][H:2 * H],
        jnp.zeros((H, 3), f32),
    ], axis=1)
    bh = jnp.concatenate([p['bout_n'], jnp.zeros((7,), f32)]).reshape(1, 8)
    hp = _headproj(x, wh, bh)
    node_pred = hp[:, 0:1]

    et_t = _eterm(ea['t'], jnp.pad(p['Wout_et'][2 * H:], ((0, 0), (0, 7))),
                  jnp.pad(p['bout_et'], (0, 7)).reshape(1, 8))[:, 0]
    et_v = _eterm(ea['v'], jnp.pad(p['Wout_ev'][2 * H:], ((0, 0), (0, 7))),
                  jnp.pad(p['bout_ev'], (0, 7)).reshape(1, 8))[:, 0]

    ep = _edge_heads(
        src['t'].reshape(NCHUNK, 128), dst['t'].reshape(NCHUNK, 128),
        et_t.reshape(NCHUNK, 128), hp[:, 1], hp[:, 2],
        src['v'].reshape(NCHUNK, 128), dst['v'].reshape(NCHUNK, 128),
        et_v.reshape(NCHUNK, 128), hp[:, 3], hp[:, 4])
    ep_t = ep[0].reshape(E, 1)
    ep_v = ep[1].reshape(E, 1)
    return (node_pred, ep_t, ep_v)


# speedup vs baseline: 1.6934x; 1.0691x over previous
import functools

import jax
import jax.numpy as jnp
from jax import lax
from jax.experimental import pallas as pl
from jax.experimental.pallas import tpu as pltpu
from jax.experimental.pallas import tpu_sc as plsc

N = 10000
E = 160000
H = 128
f32 = jnp.float32
bf16 = jnp.bfloat16


BN = 1000
BE = 640
NCHUNK = E // 128


BP = 2000


def _proj_body(x_ref, w_ref, o_ref):
    o_ref[0] = jnp.dot(x_ref[...], w_ref[0], preferred_element_type=f32)


def _proj(x, w_stack):
    k = w_stack.shape[0]
    return pl.pallas_call(
        _proj_body,
        grid=(N // BP, k),
        in_specs=[
            pl.BlockSpec((BP, H), lambda i, j: (i, 0)),
            pl.BlockSpec((1, H, H), lambda i, j: (j, 0, 0)),
        ],
        out_specs=pl.BlockSpec((1, BP, H), lambda i, j: (j, i, 0)),
        out_shape=jax.ShapeDtypeStruct((k, N, H), f32),
    )(x, w_stack)


def _msg_body(g_ref, ea_ref, w1_ref, b1_ref, w2_ref, b2_ref, o_ref):
    h = g_ref[0].astype(f32) + jnp.dot(ea_ref[...].astype(bf16), w1_ref[...],
                                       preferred_element_type=f32)
    h = jnp.maximum(h + b1_ref[...], 0.0)
    o_ref[...] = jnp.dot(h.astype(bf16), w2_ref[...],
                         preferred_element_type=f32) + b2_ref[...]


def _msg(g9, kplane, ea, w1, b1, w2, b2):
    return pl.pallas_call(
        _msg_body,
        grid=(E // BE,),
        in_specs=[
            pl.BlockSpec((1, BE, H), lambda i: (kplane, i, 0)),
            pl.BlockSpec((BE, H), lambda i: (i, 0)),
            pl.BlockSpec((H, H), lambda i: (0, 0)),
            pl.BlockSpec((1, H), lambda i: (0, 0)),
            pl.BlockSpec((H, H), lambda i: (0, 0)),
            pl.BlockSpec((1, H), lambda i: (0, 0)),
        ],
        out_specs=pl.BlockSpec((BE, H), lambda i: (i, 0)),
        out_shape=jax.ShapeDtypeStruct((EPAD, H), f32),
    )(g9, ea, w1, b1, w2, b2)


def _edge_body(gs_ref, gd_ref, ea_ref, w1_ref, b1_ref, w2_ref, b2_ref, o_ref):
    ea = ea_ref[...]
    h = (gs_ref[0] + gd_ref[0]).astype(f32) + jnp.dot(
        ea.astype(bf16), w1_ref[...], preferred_element_type=f32)
    h = jnp.maximum(h + b1_ref[...], 0.0)
    o_ref[...] = ea + jnp.dot(h.astype(bf16), w2_ref[...],
                              preferred_element_type=f32) + b2_ref[...]


def _edge(g9, ks, kd, ea, w1, b1, w2, b2):
    return pl.pallas_call(
        _edge_body,
        grid=(E // BE,),
        in_specs=[
            pl.BlockSpec((1, BE, H), lambda i: (ks, i, 0)),
            pl.BlockSpec((1, BE, H), lambda i: (kd, i, 0)),
            pl.BlockSpec((BE, H), lambda i: (i, 0)),
            pl.BlockSpec((H, H), lambda i: (0, 0)),
            pl.BlockSpec((1, H), lambda i: (0, 0)),
            pl.BlockSpec((H, H), lambda i: (0, 0)),
            pl.BlockSpec((1, H), lambda i: (0, 0)),
        ],
        out_specs=pl.BlockSpec((BE, H), lambda i: (i, 0)),
        out_shape=jax.ShapeDtypeStruct((E, H), f32),
    )(g9, g9, ea, w1, b1, w2, b2)


def _node_body(x_ref, at_ref, as_ref, av_ref, w_ref, b1_ref, w2_ref, b2_ref, o_ref):
    x = x_ref[...]
    u = jnp.dot(x, w_ref[0], preferred_element_type=f32)
    u += jnp.dot(at_ref[0] + at_ref[1], w_ref[1], preferred_element_type=f32)
    u += jnp.dot(as_ref[0] + as_ref[1], w_ref[2], preferred_element_type=f32)
    u += jnp.dot(av_ref[0] + av_ref[1], w_ref[3], preferred_element_type=f32)
    u = jnp.maximum(u + b1_ref[...], 0.0)
    o_ref[...] = x + jnp.dot(u, w2_ref[...], preferred_element_type=f32) + b2_ref[...]


def _node(x, agg_t, agg_s, agg_v, w1s, b1, w2, b2):
    return pl.pallas_call(
        _node_body,
        grid=(N // BN,),
        in_specs=[
            pl.BlockSpec((BN, H), lambda i: (i, 0)),
            pl.BlockSpec((2, BN, H), lambda i: (0, i, 0)),
            pl.BlockSpec((2, BN, H), lambda i: (0, i, 0)),
            pl.BlockSpec((2, BN, H), lambda i: (0, i, 0)),
            pl.BlockSpec((4, H, H), lambda i: (0, 0, 0)),
            pl.BlockSpec((1, H), lambda i: (0, 0)),
            pl.BlockSpec((H, H), lambda i: (0, 0)),
            pl.BlockSpec((1, H), lambda i: (0, 0)),
        ],
        out_specs=pl.BlockSpec((BN, H), lambda i: (i, 0)),
        out_shape=jax.ShapeDtypeStruct((N, H), f32),
    )(x, agg_t, agg_s, agg_v, w1s, b1, w2, b2)


def _headproj_body(x_ref, w_ref, b_ref, o_ref):
    o_ref[...] = jnp.dot(x_ref[...], w_ref[...], preferred_element_type=f32) + b_ref[...]


def _headproj(x, w, b):
    return pl.pallas_call(
        _headproj_body,
        grid=(N // BN,),
        in_specs=[
            pl.BlockSpec((BN, H), lambda i: (i, 0)),
            pl.BlockSpec((H, 8), lambda i: (0, 0)),
            pl.BlockSpec((1, 8), lambda i: (0, 0)),
        ],
        out_specs=pl.BlockSpec((BN, 8), lambda i: (i, 0)),
        out_shape=jax.ShapeDtypeStruct((N, 8), f32),
    )(x, w, b)


def _eterm(ea, w, b):
    return pl.pallas_call(
        _headproj_body,
        grid=(E // BE,),
        in_specs=[
            pl.BlockSpec((BE, H), lambda i: (i, 0)),
            pl.BlockSpec((H, 8), lambda i: (0, 0)),
            pl.BlockSpec((1, 8), lambda i: (0, 0)),
        ],
        out_specs=pl.BlockSpec((BE, 8), lambda i: (i, 0)),
        out_shape=jax.ShapeDtypeStruct((E, 8), f32),
    )(ea, w, b)


@functools.lru_cache(maxsize=1)
def _mesh():
    return plsc.VectorSubcoreMesh(core_axis_name="core", subcore_axis_name="subcore")


def _gather_rows(table, idx2d, m):
    @functools.partial(
        pl.kernel,
        out_type=jax.ShapeDtypeStruct((m, H), table.dtype),
        mesh=_mesh(),
        compiler_params=pltpu.CompilerParams(use_tc_tiling_on_sc=False),
    )
    def k(table_hbm, i_hbm, o_hbm):
        def body(i_vmem, o_vmem):
            pltpu.sync_copy(table_hbm.at[i_vmem.at[0]], o_vmem)

        pltpu.emit_pipeline(
            body,
            grid=(m // 128,),
            in_specs=[pl.BlockSpec((1, 128), lambda i: (0, i))],
            out_specs=[pl.BlockSpec((128, H), lambda i: (i, 0))],
            core_axis_name=("core", "subcore"),
            dimension_semantics=(pltpu.PARALLEL,),
        )(i_hbm, o_hbm)

    return k(table, idx2d)


NCHUNK_PAD = 1280
EPAD = NCHUNK_PAD * 128
ACC_N = N + 16


def _segsum(msg, dstm, zrows):
    @functools.partial(
        pl.kernel,
        out_type=jax.ShapeDtypeStruct((2, N, H), f32),
        mesh=_mesh(),
        compiler_params=pltpu.CompilerParams(use_tc_tiling_on_sc=False),
        scratch_types=[
            pltpu.VMEM_SHARED((ACC_N, H), f32),
            pltpu.VMEM((2, 1, 128), jnp.int32),
            pltpu.VMEM((2, 128, H), f32),
            pltpu.SemaphoreType.DMA,
            pltpu.SemaphoreType.DMA,
            pltpu.SemaphoreType.DMA,
            pltpu.SemaphoreType.DMA,
        ],
    )
    def k(msg_hbm, dstm_hbm, z_hbm, out_hbm, acc, idx_b, rows_b,
          si0, si1, sr0, sr1):
        cid = lax.axis_index("core")
        sid = lax.axis_index("subcore")
        zstripe = ACC_N // 16
        pltpu.sync_copy(z_hbm, acc.at[pl.ds(sid * zstripe, zstripe)])
        plsc.subcore_barrier()
        base = (cid * 16 + sid) * 40
        sems = (si0, si1)
        semr = (sr0, sr1)

        def issue(c, slot):
            pltpu.async_copy(dstm_hbm.at[pl.ds(c, 1)], idx_b.at[slot],
                             sems[slot])
            pltpu.async_copy(msg_hbm.at[pl.ds(c * 128, 128)], rows_b.at[slot],
                             semr[slot])

        def drain_scatter(c_next, slot, more):
            pltpu.make_async_copy(dstm_hbm.at[pl.ds(0, 1)], idx_b.at[slot],
                                  sems[slot]).wait()
            pltpu.make_async_copy(msg_hbm.at[pl.ds(0, 128)], rows_b.at[slot],
                                  semr[slot]).wait()
            pltpu.sync_copy(rows_b.at[slot], acc.at[idx_b.at[slot, 0]],
                            add=True)

            @pl.when(more)
            def _():
                issue(c_next, slot)

        issue(base, 0)
        issue(base + 1, 1)

        def pair(pp, _):
            drain_scatter(base + 2 * pp + 2, 0, pp < 19)
            drain_scatter(base + 2 * pp + 3, 1, pp < 19)
            return 0

        lax.fori_loop(0, 20, pair, 0)
        plsc.subcore_barrier()
        stripe = N // 16
        pltpu.sync_copy(
            acc.at[pl.ds(sid * stripe, stripe)],
            out_hbm.at[cid, pl.ds(sid * stripe, stripe)],
        )

    return k(msg, dstm, zrows)


def _edge_heads(srcm_t, dstm_t, et_t, tab_st, tab_dt,
                srcm_v, dstm_v, et_v, tab_sv, tab_dv):
    @functools.partial(
        pl.kernel,
        out_type=jax.ShapeDtypeStruct((2, NCHUNK, 128), f32),
        mesh=_mesh(),
        compiler_params=pltpu.CompilerParams(use_tc_tiling_on_sc=False,
                                             needs_layout_passes=False),
        scratch_types=[
            pltpu.VMEM((N,), f32),
            pltpu.VMEM((N,), f32),
            pltpu.VMEM((N,), f32),
            pltpu.VMEM((N,), f32),
            pltpu.VMEM((1, 128), jnp.int32),
            pltpu.VMEM((1, 128), jnp.int32),
            pltpu.VMEM((1, 128), f32),
            pltpu.VMEM((1, 128), f32),
        ],
    )
    def k(srcm_t_hbm, dstm_t_hbm, et_t_hbm, tab_st_hbm, tab_dt_hbm,
          srcm_v_hbm, dstm_v_hbm, et_v_hbm, tab_sv_hbm, tab_dv_hbm,
          out_hbm, st_v, dt_v, sv_v, dv_v, is_b, id_b, e_b, o_b):
        cid = lax.axis_index("core")
        sid = lax.axis_index("subcore")
        wid = sid * 2 + cid
        pltpu.sync_copy(tab_st_hbm, st_v)
        pltpu.sync_copy(tab_dt_hbm, dt_v)
        pltpu.sync_copy(tab_sv_hbm, sv_v)
        pltpu.sync_copy(tab_dv_hbm, dv_v)
        nk = jnp.where(wid < NCHUNK % 32, NCHUNK // 32 + 1, NCHUNK // 32)

        for ty, (s_hbm, d_hbm, e_hbm, stab, dtab) in enumerate(
            [(srcm_t_hbm, dstm_t_hbm, et_t_hbm, st_v, dt_v),
             (srcm_v_hbm, dstm_v_hbm, et_v_hbm, sv_v, dv_v)]):

            def step(kk, _):
                c = wid + kk * 32
                pltpu.sync_copy(s_hbm.at[pl.ds(c, 1)], is_b)
                pltpu.sync_copy(d_hbm.at[pl.ds(c, 1)], id_b)
                pltpu.sync_copy(e_hbm.at[pl.ds(c, 1)], e_b)
                for j in range(8):
                    sl = pl.ds(j * 16, 16)
                    vs = plsc.load_gather(stab, [is_b[0, sl]])
                    vd = plsc.load_gather(dtab, [id_b[0, sl]])
                    o_b[0, sl] = vs + vd + e_b[0, sl]
                pltpu.sync_copy(o_b, out_hbm.at[ty, pl.ds(c, 1)])
                return 0

            lax.fori_loop(0, nk, step, 0)

    return k(srcm_t, dstm_t, et_t, tab_st, tab_dt,
             srcm_v, dstm_v, et_v, tab_sv, tab_dv)


def kernel(x, edge_attr_t, edge_attr_s, edge_attr_v, params,
           edge_index_t, edge_index_s, edge_index_v):
    p = params
    ea = {'t': edge_attr_t, 's': edge_attr_s, 'v': edge_attr_v}
    ei = {'t': edge_index_t, 's': edge_index_s, 'v': edge_index_v}
    ets = ['t', 's', 'v']

    w9 = jnp.stack(
        [p['Wm_' + et + '1'][:H] for et in ets]
        + sum([[p['We_' + et + '1'][:H], p['We_' + et + '1'][H:2 * H]] for et in ets], []),
        axis=0)
    wn1s = jnp.stack([p['Wn1'][i * H:(i + 1) * H] for i in range(4)], axis=0)

    src = {et: ei[et][0] for et in ets}
    dst = {et: ei[et][1] for et in ets}
    gidx_m = jnp.concatenate([
        src['t'], src['s'] + N, src['v'] + 2 * N,
    ]).reshape(1, 3 * E)
    gidx_e = jnp.concatenate([
        src['t'] + 3 * N, dst['t'] + 4 * N,
        src['s'] + 5 * N, dst['s'] + 6 * N,
        src['v'] + 7 * N, dst['v'] + 8 * N,
    ]).reshape(1, 6 * E)
    pad = jnp.full((EPAD - E,), N, jnp.int32)
    dstm = {et: jnp.concatenate([dst[et], pad]).reshape(NCHUNK_PAD, 128)
            for et in ets}
    zrows = jnp.zeros((ACC_N // 16, H), f32)

    b1 = {et: p['bm_' + et + '1'].reshape(1, H) for et in ets}
    b2 = {et: p['bm_' + et + '2'].reshape(1, H) for et in ets}
    be1 = {et: p['be_' + et + '1'].reshape(1, H) for et in ets}
    be2 = {et: p['be_' + et + '2'].reshape(1, H) for et in ets}
    bn1 = p['bn1'].reshape(1, H)
    bn2 = p['bn2'].reshape(1, H)

    for _ in range(2):
        proj = _proj(x, w9)
        table = proj.reshape(9 * N, H)
        g3 = _gather_rows(table, gidx_m, 3 * E).reshape(3, E, H)
        g6 = _gather_rows(table, gidx_e, 6 * E).reshape(6, E, H)
        aggs = []
        for i, et in enumerate(ets):
            msg = _msg(g3, i, ea[et], p['Wm_' + et + '1'][H:].astype(bf16),
                       b1[et], p['Wm_' + et + '2'].astype(bf16), b2[et])
            aggs.append(_segsum(msg, dstm[et], zrows))
        x_new = _node(x, aggs[0], aggs[1], aggs[2], wn1s, bn1, p['Wn2'], bn2)
        for i, et in enumerate(ets):
            ea[et] = _edge(g6, 2 * i, 2 * i + 1, ea[et],
                           p['We_' + et + '1'][2 * H:].astype(bf16), be1[et],
                           p['We_' + et + '2'].astype(bf16), be2[et])
        x = x_new

    wh = jnp.concatenate([
        p['Wout_n'], p['Wout_et'][:H], p['Wout_et'][H:2 * H],
        p['Wout_ev'][:H], p['Wout_ev'][H:2 * H],
        jnp.zeros((H, 3), f32),
    ], axis=1)
    bh = jnp.concatenate([p['bout_n'], jnp.zeros((7,), f32)]).reshape(1, 8)
    hp = _headproj(x, wh, bh)
    node_pred = hp[:, 0:1]

    et_t = _eterm(ea['t'], jnp.pad(p['Wout_et'][2 * H:], ((0, 0), (0, 7))),
                  jnp.pad(p['bout_et'], (0, 7)).reshape(1, 8))[:, 0]
    et_v = _eterm(ea['v'], jnp.pad(p['Wout_ev'][2 * H:], ((0, 0), (0, 7))),
                  jnp.pad(p['bout_ev'], (0, 7)).reshape(1, 8))[:, 0]

    ep = _edge_heads(
        src['t'].reshape(NCHUNK, 128), dst['t'].reshape(NCHUNK, 128),
        et_t.reshape(NCHUNK, 128), hp[:, 1], hp[:, 2],
        src['v'].reshape(NCHUNK, 128), dst['v'].reshape(NCHUNK, 128),
        et_v.reshape(NCHUNK, 128), hp[:, 3], hp[:, 4])
    ep_t = ep[0].reshape(E, 1)
    ep_v = ep[1].reshape(E, 1)
    return (node_pred, ep_t, ep_v)
